# baseline (speedup 1.0000x reference)
import functools
import math

import numpy as np
import jax
import jax.numpy as jnp
from jax import lax
from jax.experimental import pallas as pl
from jax.experimental.pallas import tpu as pltpu
from jax.experimental.pallas import tpu_sc as plsc

_NUM_FIELDS = 26
_NUM_GROUPS = 4
_BASE_DIM = 128
_BATCH = 16384
_FIELD_DIM = 100000


def _group_structure():
    np.random.seed(0)
    dim_indices = np.arange(_NUM_FIELDS)
    np.random.shuffle(dim_indices)
    chunk = _NUM_FIELDS // _NUM_GROUPS
    groups = []
    for i in range(_NUM_GROUPS):
        if i == _NUM_GROUPS - 1:
            groups.append(dim_indices[i * chunk:])
            break
        groups.append(dim_indices[i * chunk:(i + 1) * chunk])
    total = _NUM_FIELDS * _FIELD_DIM
    emb_dims = []
    for g in groups:
        div = total / (len(g) * _FIELD_DIM)
        emb_dims.append(max(2, int(_BASE_DIM / 2 ** int(math.log2(div)))))
    return groups, emb_dims


_GROUPS, _EMB_DIMS = _group_structure()
_NFIELDS = [len(g) for g in _GROUPS]
_VOCABS = [n * _FIELD_DIM for n in _NFIELDS]

_NC, _NS, _L = 2, 16, 16
_NW = _NC * _NS
_BPW = _BATCH // _NW
_CHUNK = 128
_NCHUNK = _BPW // _CHUNK
_VBLK = 1024
_RBLK = 32768


def _repack_body(d, x_ref, o_ref):
    npk = 128 // d
    seg = _VBLK // npk
    for u in range(_RBLK // _VBLK):
        acc = None
        for k in range(npk):
            sl = slice(u * _VBLK + k * seg, u * _VBLK + (k + 1) * seg)
            part = lax.dot_general(
                x_ref[:, sl], jnp.eye(d, 128, k=d * k, dtype=jnp.float32),
                (((0,), (0,)), ((), ())),
                preferred_element_type=jnp.float32)
            acc = part if acc is None else acc + part
        o_ref[u] = acc.astype(jnp.bfloat16)


def _repack(embT, d, vocab):
    nblk = (vocab + _VBLK - 1) // _VBLK
    ngrid = (vocab + _RBLK - 1) // _RBLK
    nblk_pad = ngrid * (_RBLK // _VBLK)
    rows = _VBLK // (128 // d)
    out = pl.pallas_call(
        functools.partial(_repack_body, d),
        grid=(ngrid,),
        in_specs=[pl.BlockSpec((d, _RBLK), lambda i: (0, i))],
        out_specs=pl.BlockSpec(
            (_RBLK // _VBLK, rows, 128), lambda i: (i, 0, 0)),
        out_shape=jax.ShapeDtypeStruct((nblk_pad, rows, 128), jnp.bfloat16),
    )(embT)
    del nblk
    return out.reshape(nblk_pad * _VBLK, d)


def _pack_index(t, d):
    b = (t >> 10) << 10
    if d == 32:
        return b + ((t & 255) << 2) + ((t >> 8) & 3)
    else:
        return b + ((t & 511) << 1) + ((t >> 9) & 1)


def _sc_group_body(n, d, offs, idx_hbm, tab, out, idxbs, gbufs, acc, sem):
    wid = lax.axis_index("s") * _NC + lax.axis_index("c")

    def chunk_body(ch, carry):
        base = wid * _BPW + ch * _CHUNK
        for j in range(n):
            pltpu.sync_copy(
                idx_hbm.at[pl.ds(j * _BATCH + base, _CHUNK)], idxbs[j])
        for j in range(n):
            for i in range(_CHUNK // _L):
                sl = pl.ds(i * _L, _L)
                idxbs[j][sl] = _pack_index(idxbs[j][sl] + offs[j], d)
        copies = [
            pltpu.async_copy(tab.at[idxbs[j]], gbufs[j], sem)
            for j in range(n)
        ]
        for c in copies:
            c.wait()
        nb = d // (2 * _L)

        def add_body(r, c):
            for k2 in range(nb):
                sl = pl.ds(k2 * 2 * _L, 2 * _L)
                v = gbufs[0][r, sl]
                for j in range(1, n):
                    v = v + gbufs[j][r, sl]
                acc[r, sl] = v
            return c

        lax.fori_loop(0, _CHUNK, add_body, 0, unroll=4)
        pltpu.sync_copy(acc, out.at[pl.ds(base, _CHUNK), :])
        return carry

    lax.fori_loop(0, _NCHUNK, chunk_body, 0)


def _sc_group(idx_flat, packed_tab, n, d, offs):
    mesh = plsc.VectorSubcoreMesh(
        core_axis_name="c", subcore_axis_name="s",
        num_cores=_NC, num_subcores=_NS)
    kern = pl.kernel(
        functools.partial(_sc_group_body, n, d, offs),
        out_type=jax.ShapeDtypeStruct((_BATCH, d), jnp.bfloat16),
        mesh=mesh,
        scratch_types=[
            [pltpu.VMEM((_CHUNK,), jnp.int32) for _ in range(n)],
            [pltpu.VMEM((_CHUNK, d), jnp.bfloat16) for _ in range(n)],
            pltpu.VMEM((_CHUNK, d), jnp.bfloat16),
            pltpu.SemaphoreType.DMA,
        ],
        compiler_params=pltpu.CompilerParams(use_tc_tiling_on_sc=False),
    )
    return kern(idx_flat, packed_tab)


def _mm_body(x0_ref, x1_ref, x2_ref, x3_ref, w_ref, o_ref):
    x = jnp.concatenate(
        [x0_ref[...], x1_ref[...], x2_ref[...], x3_ref[...]], axis=1)
    o_ref[...] = jnp.dot(x, w_ref[...], preferred_element_type=jnp.float32)


def _project(pooled, w):
    blk = 2048
    pdim = sum(_EMB_DIMS)
    return pl.pallas_call(
        _mm_body,
        grid=(_BATCH // blk,),
        in_specs=[
            pl.BlockSpec((blk, d), lambda i: (i, 0)) for d in _EMB_DIMS
        ] + [
            pl.BlockSpec((pdim, _BASE_DIM), lambda i: (0, 0)),
        ],
        out_specs=pl.BlockSpec((blk, _BASE_DIM), lambda i: (i, 0)),
        out_shape=jax.ShapeDtypeStruct((_BATCH, _BASE_DIM), jnp.float32),
    )(*pooled, w)


def kernel(input_, embed_w_0, linear_w_0, embed_w_1, linear_w_1,
           embed_w_2, linear_w_2, embed_w_3, linear_w_3):
    embed_ws = [embed_w_0, embed_w_1, embed_w_2, embed_w_3]
    pooled = []
    for g in range(_NUM_GROUPS):
        n, d, vocab = _NFIELDS[g], _EMB_DIMS[g], _VOCABS[g]
        packed = _repack(embed_ws[g].T, d, vocab)
        cols = np.asarray(_GROUPS[g], dtype=np.int32)
        idx_flat = jnp.transpose(input_[:, cols]).reshape(-1)
        offs = [f * _FIELD_DIM for f in range(n)]
        pooled.append(_sc_group(idx_flat, packed, n, d, offs))
    w = jnp.concatenate([linear_w_0.T, linear_w_1.T,
                         linear_w_2.T, linear_w_3.T],
                        axis=0).astype(jnp.bfloat16)
    return _project(pooled, w)


# speedup vs baseline: 2.5483x; 2.5483x over previous
import functools
import math

import numpy as np
import jax
import jax.numpy as jnp
from jax import lax
from jax.experimental import pallas as pl
from jax.experimental.pallas import tpu as pltpu
from jax.experimental.pallas import tpu_sc as plsc

_NUM_FIELDS = 26
_NUM_GROUPS = 4
_BASE_DIM = 128
_BATCH = 16384
_FIELD_DIM = 100000


def _group_structure():
    np.random.seed(0)
    dim_indices = np.arange(_NUM_FIELDS)
    np.random.shuffle(dim_indices)
    chunk = _NUM_FIELDS // _NUM_GROUPS
    groups = []
    for i in range(_NUM_GROUPS):
        if i == _NUM_GROUPS - 1:
            groups.append(dim_indices[i * chunk:])
            break
        groups.append(dim_indices[i * chunk:(i + 1) * chunk])
    total = _NUM_FIELDS * _FIELD_DIM
    emb_dims = []
    for g in groups:
        div = total / (len(g) * _FIELD_DIM)
        emb_dims.append(max(2, int(_BASE_DIM / 2 ** int(math.log2(div)))))
    return groups, emb_dims


_GROUPS, _EMB_DIMS = _group_structure()
_NFIELDS = [len(g) for g in _GROUPS]
_VOCABS = [n * _FIELD_DIM for n in _NFIELDS]

_NC, _NS, _L = 2, 16, 16
_NW = _NC * _NS
_BPW = _BATCH // _NW
_CHUNK = 128
_NCHUNK = _BPW // _CHUNK
_VBLK = 1024
_RBLK = 32768


def _repack_body(d, x_ref, o_ref):
    npk = 128 // d
    seg = _VBLK // npk
    for u in range(_RBLK // _VBLK):
        acc = None
        for k in range(npk):
            sl = slice(u * _VBLK + k * seg, u * _VBLK + (k + 1) * seg)
            part = lax.dot_general(
                x_ref[:, sl], jnp.eye(d, 128, k=d * k, dtype=jnp.float32),
                (((0,), (0,)), ((), ())),
                preferred_element_type=jnp.float32)
            acc = part if acc is None else acc + part
        o_ref[u] = acc


def _repack(embT, d, vocab):
    nblk = (vocab + _VBLK - 1) // _VBLK
    ngrid = (vocab + _RBLK - 1) // _RBLK
    nblk_pad = ngrid * (_RBLK // _VBLK)
    rows = _VBLK // (128 // d)
    out = pl.pallas_call(
        functools.partial(_repack_body, d),
        grid=(ngrid,),
        in_specs=[pl.BlockSpec((d, _RBLK), lambda i: (0, i))],
        out_specs=pl.BlockSpec(
            (_RBLK // _VBLK, rows, 128), lambda i: (i, 0, 0)),
        out_shape=jax.ShapeDtypeStruct((nblk_pad, rows, 128), jnp.float32),
    )(embT)
    del nblk
    return out.reshape(nblk_pad * _VBLK, d)


def _pack_index(t, d):
    b = (t >> 10) << 10
    if d == 32:
        return b + ((t & 255) << 2) + ((t >> 8) & 3)
    else:
        return b + ((t & 511) << 1) + ((t >> 9) & 1)


def _sc_group_body(n, d, offs, idx_hbm, tab, out, idxbs, gbufs, acc, sem):
    wid = lax.axis_index("s") * _NC + lax.axis_index("c")

    def chunk_body(ch, carry):
        base = wid * _BPW + ch * _CHUNK
        for j in range(n):
            pltpu.sync_copy(
                idx_hbm.at[pl.ds(j * _BATCH + base, _CHUNK)], idxbs[j])
        for j in range(n):
            for i in range(_CHUNK // _L):
                sl = pl.ds(i * _L, _L)
                idxbs[j][sl] = _pack_index(idxbs[j][sl] + offs[j], d)
        copies = [
            pltpu.async_copy(tab.at[idxbs[j]], gbufs[j], sem)
            for j in range(n)
        ]
        for c in copies:
            c.wait()
        nb = d // _L

        def add_body(r, c):
            for k2 in range(nb):
                sl = pl.ds(k2 * _L, _L)
                v = gbufs[0][r, sl]
                for j in range(1, n):
                    v = v + gbufs[j][r, sl]
                acc[r, sl] = v
            return c

        lax.fori_loop(0, _CHUNK, add_body, 0, unroll=4)
        pltpu.sync_copy(acc, out.at[pl.ds(base, _CHUNK), :])
        return carry

    lax.fori_loop(0, _NCHUNK, chunk_body, 0)


def _sc_group(idx_flat, packed_tab, n, d, offs):
    mesh = plsc.VectorSubcoreMesh(
        core_axis_name="c", subcore_axis_name="s",
        num_cores=_NC, num_subcores=_NS)
    kern = pl.kernel(
        functools.partial(_sc_group_body, n, d, offs),
        out_type=jax.ShapeDtypeStruct((_BATCH, d), jnp.float32),
        mesh=mesh,
        scratch_types=[
            [pltpu.VMEM((_CHUNK,), jnp.int32) for _ in range(n)],
            [pltpu.VMEM((_CHUNK, d), jnp.float32) for _ in range(n)],
            pltpu.VMEM((_CHUNK, d), jnp.float32),
            pltpu.SemaphoreType.DMA,
        ],
        compiler_params=pltpu.CompilerParams(use_tc_tiling_on_sc=False),
    )
    return kern(idx_flat, packed_tab)


def _mm_body(x0_ref, x1_ref, x2_ref, x3_ref, w_ref, o_ref):
    x = jnp.concatenate(
        [x0_ref[...], x1_ref[...], x2_ref[...], x3_ref[...]], axis=1)
    o_ref[...] = jnp.dot(x, w_ref[...], preferred_element_type=jnp.float32)


def _project(pooled, w):
    blk = 2048
    pdim = sum(_EMB_DIMS)
    return pl.pallas_call(
        _mm_body,
        grid=(_BATCH // blk,),
        in_specs=[
            pl.BlockSpec((blk, d), lambda i: (i, 0)) for d in _EMB_DIMS
        ] + [
            pl.BlockSpec((pdim, _BASE_DIM), lambda i: (0, 0)),
        ],
        out_specs=pl.BlockSpec((blk, _BASE_DIM), lambda i: (i, 0)),
        out_shape=jax.ShapeDtypeStruct((_BATCH, _BASE_DIM), jnp.float32),
    )(*pooled, w)


def kernel(input_, embed_w_0, linear_w_0, embed_w_1, linear_w_1,
           embed_w_2, linear_w_2, embed_w_3, linear_w_3):
    embed_ws = [embed_w_0, embed_w_1, embed_w_2, embed_w_3]
    pooled = []
    for g in range(_NUM_GROUPS):
        n, d, vocab = _NFIELDS[g], _EMB_DIMS[g], _VOCABS[g]
        packed = _repack(embed_ws[g].T, d, vocab)
        cols = np.asarray(_GROUPS[g], dtype=np.int32)
        idx_flat = jnp.transpose(input_[:, cols]).reshape(-1)
        offs = [f * _FIELD_DIM for f in range(n)]
        pooled.append(_sc_group(idx_flat, packed, n, d, offs))
    w = jnp.concatenate([linear_w_0.T, linear_w_1.T,
                         linear_w_2.T, linear_w_3.T],
                        axis=0)
    return _project(pooled, w)
